# Initial kernel scaffold; baseline (speedup 1.0000x reference)
import jax
import jax.numpy as jnp
from jax.experimental import pallas as pl


def kernel(x, graph_idxs, graph_mask):
    raise NotImplementedError("write your pallas kernel here")


# speedup vs baseline: 2.3352x; 2.3352x over previous
import functools

import jax
import jax.numpy as jnp
from jax import lax
from jax.experimental import pallas as pl
from jax.experimental.pallas import tpu as pltpu, tpu_sc as plsc

_KEEP_RATE = 0.9

_NC, _NS, _L = 2, 16, 16
_NW = _NC * _NS


def _sc_gather(idx_flat, mask_flat, n_per_batch, n_graphs):
    tok = idx_flat.shape[0]
    tpw = tok // _NW
    mesh = plsc.VectorSubcoreMesh(
        core_axis_name="c", subcore_axis_name="s",
        num_cores=_NC, num_subcores=_NS)

    @functools.partial(
        pl.kernel,
        out_type=jax.ShapeDtypeStruct((tok,), jnp.float32),
        mesh=mesh,
        scratch_types=[
            pltpu.VMEM((tpw,), jnp.int32),
            pltpu.VMEM((mask_flat.shape[0],), jnp.float32),
            pltpu.VMEM((tpw,), jnp.float32),
        ],
        compiler_params=pltpu.CompilerParams(needs_layout_passes=False),
    )
    def k(idx_hbm, mask_hbm, out_hbm, idx_v, mask_v, out_v):
        wid = lax.axis_index("s") * _NC + lax.axis_index("c")
        base = wid * tpw
        table_off = (base // n_per_batch) * n_graphs
        pltpu.sync_copy(mask_hbm, mask_v)
        pltpu.sync_copy(idx_hbm.at[pl.ds(base, tpw)], idx_v)

        def body(i, carry):
            sl = pl.ds(i * _L, _L)
            out_v[sl] = plsc.load_gather(mask_v, [idx_v[sl] + table_off])
            return carry

        lax.fori_loop(0, tpw // _L, body, 0)
        pltpu.sync_copy(out_v, out_hbm.at[pl.ds(base, tpw)])

    return k(idx_flat, mask_flat)


def _tc_scale(x2, scale2):
    tok, d = x2.shape
    blk = 1024
    inv_keep = 1.0 / _KEEP_RATE

    def body(x_ref, s_ref, o_ref):
        o_ref[...] = x_ref[...] * (s_ref[...] * inv_keep)

    return pl.pallas_call(
        body,
        grid=(tok // blk,),
        in_specs=[
            pl.BlockSpec((blk, d), lambda i: (i, 0)),
            pl.BlockSpec((blk, 1), lambda i: (i, 0)),
        ],
        out_specs=pl.BlockSpec((blk, d), lambda i: (i, 0)),
        out_shape=jax.ShapeDtypeStruct((tok, d), x2.dtype),
    )(x2, scale2)


def kernel(x, graph_idxs, graph_mask):
    b, n, d = x.shape
    n_graphs = graph_mask.shape[1]
    tok = b * n
    idx_flat = graph_idxs.astype(jnp.int32).reshape(tok)
    mask_flat = graph_mask.astype(jnp.float32).reshape(b * n_graphs)
    scale = _sc_gather(idx_flat, mask_flat, n, n_graphs)
    out2 = _tc_scale(x.reshape(tok, d), scale.reshape(tok, 1))
    return out2.reshape(b, n, d)
